# Initial kernel scaffold; baseline (speedup 1.0000x reference)
import jax
import jax.numpy as jnp
from jax.experimental import pallas as pl


def kernel(tokens, W_E):
    raise NotImplementedError("write your pallas kernel here")


# speedup vs baseline: 8.1864x; 8.1864x over previous
import functools

import jax
import jax.numpy as jnp
from jax import lax
from jax.experimental import pallas as pl
from jax.experimental.pallas import tpu as pltpu
from jax.experimental.pallas import tpu_sc as plsc

D_VOCAB = 100000
D_MODEL = 128
BATCH = 4096
POS = 200

_L = 128
_B = BATCH * POS
_ROWS = _B // _L
_NW = 32
_ROWS_PER_W = _ROWS // _NW
_CH = 4
_NCH = _ROWS_PER_W // _CH

_mesh = plsc.VectorSubcoreMesh(core_axis_name="c", subcore_axis_name="s")


@functools.partial(
    pl.kernel,
    mesh=_mesh,
    out_type=jax.ShapeDtypeStruct((_ROWS, _L, D_MODEL), jnp.float32),
    scratch_types=[
        pltpu.VMEM((_CH, _L), jnp.int32),
        pltpu.VMEM((_CH, _L, D_MODEL), jnp.float32),
        pltpu.SemaphoreType.DMA,
    ],
)
def _embed(table_hbm, idx_hbm, out_hbm, idx_v, rows_v, sem):
    wid = lax.axis_index("s") * 2 + lax.axis_index("c")
    row0 = wid * _ROWS_PER_W

    def chunk(c, carry):
        base = row0 + c * _CH
        pltpu.sync_copy(idx_hbm.at[pl.ds(base, _CH)], idx_v)
        copies = [
            pltpu.async_copy(table_hbm.at[idx_v.at[j]], rows_v.at[j], sem)
            for j in range(_CH)
        ]
        for cp in copies:
            cp.wait()
        pltpu.sync_copy(rows_v, out_hbm.at[pl.ds(base, _CH)])
        return carry

    lax.fori_loop(0, _NCH, chunk, 0)


def kernel(tokens, W_E):
    idx = tokens.reshape(_ROWS, _L)
    out = _embed(W_E, idx)
    return out.reshape(BATCH, POS, D_MODEL)


# speedup vs baseline: 9.2139x; 1.1255x over previous
import functools

import jax
import jax.numpy as jnp
from jax import lax
from jax.experimental import pallas as pl
from jax.experimental.pallas import tpu as pltpu
from jax.experimental.pallas import tpu_sc as plsc

D_VOCAB = 100000
D_MODEL = 128
BATCH = 4096
POS = 200

_L = 128
_B = BATCH * POS
_ROWS = _B // _L
_NW = 32
_ROWS_PER_W = _ROWS // _NW
_CH = 2
_NBUF = 2
_NCH = _ROWS_PER_W // _CH
_G = _NCH // _NBUF

_mesh = plsc.VectorSubcoreMesh(core_axis_name="c", subcore_axis_name="s")


@functools.partial(
    pl.kernel,
    mesh=_mesh,
    out_type=jax.ShapeDtypeStruct((_ROWS, _L, D_MODEL), jnp.float32),
    scratch_types=[
        pltpu.VMEM((_NBUF, _CH, _L), jnp.int32),
        pltpu.VMEM((_NBUF, _CH, _L, D_MODEL), jnp.float32),
        pltpu.SemaphoreType.DMA((_NBUF,)),
        pltpu.SemaphoreType.DMA((_NBUF,)),
    ],
)
def _embed(table_hbm, idx_hbm, out_hbm, idx_v, rows_v, gsem, osem):
    wid = lax.axis_index("s") * 2 + lax.axis_index("c")
    row0 = wid * _ROWS_PER_W

    def fire_gathers(c, b):
        base = row0 + c * _CH
        pltpu.sync_copy(idx_hbm.at[pl.ds(base, _CH)], idx_v.at[b])
        return [
            pltpu.async_copy(table_hbm.at[idx_v.at[b, j]], rows_v.at[b, j],
                             gsem.at[b])
            for j in range(_CH)
        ]

    def drain_fire_out(c, b, gathers):
        for g in gathers:
            g.wait()
        base = row0 + c * _CH
        pltpu.async_copy(rows_v.at[b], out_hbm.at[pl.ds(base, _CH)],
                         osem.at[b])

    def wait_out(c, b):
        base = row0 + c * _CH
        pltpu.make_async_copy(rows_v.at[b], out_hbm.at[pl.ds(base, _CH)],
                              osem.at[b]).wait()

    pending = [fire_gathers(b, b) for b in range(_NBUF)]
    for b in range(_NBUF):
        drain_fire_out(b, b, pending[b])

    def group(g, carry):
        pending = []
        for b in range(_NBUF):
            c = g * _NBUF + b
            wait_out(c - _NBUF, b)
            pending.append(fire_gathers(c, b))
        for b in range(_NBUF):
            drain_fire_out(g * _NBUF + b, b, pending[b])
        return carry

    lax.fori_loop(1, _G, group, 0)

    for b in range(_NBUF):
        wait_out((_G - 1) * _NBUF + b, b)


def kernel(tokens, W_E):
    idx = tokens.reshape(_ROWS, _L)
    out = _embed(W_E, idx)
    return out.reshape(BATCH, POS, D_MODEL)
